# baseline (speedup 1.0000x reference)
import functools

import jax
import jax.numpy as jnp
from jax import lax
from jax.experimental import pallas as pl
from jax.experimental.pallas import tpu as pltpu
from jax.experimental.pallas import tpu_sc as plsc


def _argmin_body(z_ref, w_ref, b_ref, e_ref, ze_ref, ind_ref, *, kb, hb):
    zb = z_ref[0]
    w = w_ref[...]
    ze = lax.dot_general(w, zb, (((1,), (0,)), ((), ())),
                         preferred_element_type=jnp.float32) + b_ref[...]
    r = ze.shape[1]
    k_total = e_ref.shape[0]
    f2 = jnp.sum(ze * ze, axis=0, keepdims=True)
    ns = hb // kb

    def inner(k, carry):
        lv, li = carry
        eb = e_ref[pl.ds(k * kb, kb), :]
        s = lax.dot_general(eb, ze, (((1,), (0,)), ((), ())),
                            preferred_element_type=jnp.float32)
        e2 = jnp.sum(eb * eb, axis=1, keepdims=True)
        dist = (f2 - 2.0 * s) + e2
        m = jnp.min(dist, axis=0, keepdims=True)
        iota = lax.broadcasted_iota(jnp.int32, (kb, r), 0) + k * kb
        ci = jnp.min(jnp.where(dist == m, iota, jnp.int32(2**30)),
                     axis=0, keepdims=True)
        upd = (m < lv) | ((m == lv) & (ci < li))
        return jnp.where(upd, m, lv), jnp.where(upd, ci, li)

    def outer(h, carry):
        bv, bi = carry
        lv0 = jnp.full((1, r), jnp.inf, jnp.float32)
        li0 = jnp.full((1, r), jnp.int32(2**30))
        lv, li = lax.fori_loop(h * ns, (h + 1) * ns, inner, (lv0, li0))
        take = lv < bv
        lvq = lv.astype(jnp.bfloat16).astype(jnp.float32)
        return jnp.where(take, lvq, bv), jnp.where(take, li, bi)

    bv0 = jnp.full((1, r), jnp.inf, jnp.float32)
    bi0 = jnp.zeros((1, r), jnp.int32)
    _, bi = lax.fori_loop(0, k_total // hb, outer, (bv0, bi0))
    ze_ref[0] = ze
    ind_ref[0] = bi


def _combine_body(ze_ref, zq_ref, out_ref, diff_ref, acc_ref, *, n_total, n_cells):
    ze = ze_ref[0]
    zq = zq_ref[0]
    delta = zq - ze
    out_ref[0] = ze + delta
    part = jnp.sum(delta * delta)
    cell = pl.program_id(0) * pl.num_programs(1) + pl.program_id(1)

    @pl.when(cell == 0)
    def _():
        acc_ref[0] = part

    @pl.when(cell != 0)
    def _():
        acc_ref[0] += part

    @pl.when(cell == n_cells - 1)
    def _():
        m = acc_ref[0] / n_total
        diff_ref[...] = jnp.full((1, 1), (0.25 * m + m) * 10.0, jnp.float32)


def _sc_gather(table, idx):
    n = idx.shape[0]
    d = table.shape[1]
    info = plsc.get_sparse_core_info()
    nw = info.num_cores * info.num_subcores
    b_per_w = n // nw
    chunk = 128
    n_chunks = b_per_w // chunk
    mesh = plsc.VectorSubcoreMesh(core_axis_name="c", subcore_axis_name="s")

    @functools.partial(
        pl.kernel, mesh=mesh,
        compiler_params=pltpu.CompilerParams(use_tc_tiling_on_sc=False),
        out_type=jax.ShapeDtypeStruct((n, d), jnp.float32),
        scratch_types=[
            pltpu.VMEM((chunk,), jnp.int32),
            pltpu.VMEM((chunk, d), jnp.float32),
            pltpu.SemaphoreType.DMA,
        ],
    )
    def gather_k(table_hbm, idx_hbm, out_hbm, idx_v, rows_v, sem):
        wid = lax.axis_index("s") * info.num_cores + lax.axis_index("c")
        base = wid * b_per_w

        def step(i, _):
            off = base + i * chunk
            pltpu.sync_copy(idx_hbm.at[pl.ds(off, chunk)], idx_v)
            pltpu.async_copy(table_hbm.at[idx_v], rows_v, sem).wait()
            pltpu.sync_copy(rows_v, out_hbm.at[pl.ds(off, chunk)])
            return 0

        lax.fori_loop(0, n_chunks, step, 0)

    return gather_k(table, idx)


def kernel(z, W_proj, b_proj, embed_w):
    B, C, H, W = z.shape
    D = W_proj.shape[0]
    K = embed_w.shape[0]
    HW = H * W
    R = 256
    KB = 512
    HB = 4096

    zf = z.reshape(B, C, HW)
    bp = b_proj.reshape(D, 1)

    ze_t, ind3 = pl.pallas_call(
        functools.partial(_argmin_body, kb=KB, hb=HB),
        grid=(B, HW // R),
        in_specs=[
            pl.BlockSpec((1, C, R), lambda b, j: (b, 0, j)),
            pl.BlockSpec((D, C), lambda b, j: (0, 0)),
            pl.BlockSpec((D, 1), lambda b, j: (0, 0)),
            pl.BlockSpec((K, D), lambda b, j: (0, 0)),
        ],
        out_specs=[
            pl.BlockSpec((1, D, R), lambda b, j: (b, 0, j)),
            pl.BlockSpec((1, 1, R), lambda b, j: (b, 0, j)),
        ],
        out_shape=[
            jax.ShapeDtypeStruct((B, D, HW), jnp.float32),
            jax.ShapeDtypeStruct((B, 1, HW), jnp.int32),
        ],
    )(zf, W_proj, bp, embed_w)

    ind_flat = ind3.reshape(B * HW)
    zq_rows = _sc_gather(embed_w, ind_flat)
    zq_t = zq_rows.reshape(B, HW, D).transpose(0, 2, 1)

    n_cells = B * (HW // R)
    zq_st, diffp = pl.pallas_call(
        functools.partial(_combine_body, n_total=float(B * HW * D),
                          n_cells=n_cells),
        grid=(B, HW // R),
        in_specs=[
            pl.BlockSpec((1, D, R), lambda b, j: (b, 0, j)),
            pl.BlockSpec((1, D, R), lambda b, j: (b, 0, j)),
        ],
        out_specs=[
            pl.BlockSpec((1, D, R), lambda b, j: (b, 0, j)),
            pl.BlockSpec((1, 1), lambda b, j: (0, 0)),
        ],
        out_shape=[
            jax.ShapeDtypeStruct((B, D, HW), jnp.float32),
            jax.ShapeDtypeStruct((1, 1), jnp.float32),
        ],
        scratch_shapes=[pltpu.SMEM((1,), jnp.float32)],
    )(ze_t, zq_t)

    z_q = zq_st.reshape(B, D, H, W)
    diff = diffp.reshape(())
    ind = ind3.reshape(B, H, W)
    return z_q, diff, ind


# speedup vs baseline: 1.0620x; 1.0620x over previous
import functools

import jax
import jax.numpy as jnp
from jax import lax
from jax.experimental import pallas as pl
from jax.experimental.pallas import tpu as pltpu
from jax.experimental.pallas import tpu_sc as plsc


def _argmin_body(z_ref, w_ref, b_ref, e_ref, ze_ref, ind_ref,
                 ebd_ref, e2_ref, *, kb, hb):
    first = (pl.program_id(0) == 0) & (pl.program_id(1) == 0)

    @pl.when(first)
    def _():
        e = e_ref[...]
        ebd_ref[...] = (2.0 * e).astype(jnp.bfloat16)
        e2_ref[...] = jnp.sum(e * e, axis=1, keepdims=True)

    zb = z_ref[0]
    w = w_ref[...]
    ze = lax.dot_general(w, zb, (((1,), (0,)), ((), ())),
                         preferred_element_type=jnp.float32) + b_ref[...]
    r = ze.shape[1]
    k_total = e_ref.shape[0]
    f2 = jnp.sum(ze * ze, axis=0, keepdims=True)
    ze_bf = ze.astype(jnp.bfloat16)
    iota = lax.broadcasted_iota(jnp.int32, (kb, r), 0)
    ns = hb // kb

    def inner(k, carry):
        lv, li = carry
        eb = ebd_ref[pl.ds(k * kb, kb), :]
        s2 = lax.dot_general(eb, ze_bf, (((1,), (0,)), ((), ())),
                             preferred_element_type=jnp.float32)
        e2 = e2_ref[pl.ds(k * kb, kb), :]
        dist = (f2 - s2) + e2
        m = jnp.min(dist, axis=0, keepdims=True)
        ci = jnp.min(jnp.where(dist == m, iota, jnp.int32(2**30)),
                     axis=0, keepdims=True) + k * kb
        upd = (m < lv) | ((m == lv) & (ci < li))
        return jnp.where(upd, m, lv), jnp.where(upd, ci, li)

    def outer(h, carry):
        bv, bi = carry
        lv0 = jnp.full((1, r), jnp.inf, jnp.float32)
        li0 = jnp.full((1, r), jnp.int32(2**30))
        lv, li = lax.fori_loop(h * ns, (h + 1) * ns, inner, (lv0, li0))
        take = lv < bv
        lvq = lv.astype(jnp.bfloat16).astype(jnp.float32)
        return jnp.where(take, lvq, bv), jnp.where(take, li, bi)

    bv0 = jnp.full((1, r), jnp.inf, jnp.float32)
    bi0 = jnp.zeros((1, r), jnp.int32)
    _, bi = lax.fori_loop(0, k_total // hb, outer, (bv0, bi0))
    ze_ref[0] = ze
    ind_ref[0] = bi


def _combine_body(ze_ref, zq_ref, out_ref, diff_ref, acc_ref, *, n_total, n_cells):
    ze = ze_ref[0]
    zq = zq_ref[0]
    delta = zq - ze
    out_ref[0] = ze + delta
    part = jnp.sum(delta * delta)
    cell = pl.program_id(0) * pl.num_programs(1) + pl.program_id(1)

    @pl.when(cell == 0)
    def _():
        acc_ref[0] = part

    @pl.when(cell != 0)
    def _():
        acc_ref[0] += part

    @pl.when(cell == n_cells - 1)
    def _():
        m = acc_ref[0] / n_total
        diff_ref[...] = jnp.full((1, 1), (0.25 * m + m) * 10.0, jnp.float32)


def _sc_gather(table, idx):
    n = idx.shape[0]
    d = table.shape[1]
    info = plsc.get_sparse_core_info()
    nw = info.num_cores * info.num_subcores
    b_per_w = n // nw
    chunk = 128
    n_chunks = b_per_w // chunk
    mesh = plsc.VectorSubcoreMesh(core_axis_name="c", subcore_axis_name="s")

    @functools.partial(
        pl.kernel, mesh=mesh,
        compiler_params=pltpu.CompilerParams(use_tc_tiling_on_sc=False),
        out_type=jax.ShapeDtypeStruct((n, d), jnp.float32),
        scratch_types=[
            pltpu.VMEM((chunk,), jnp.int32),
            pltpu.VMEM((chunk, d), jnp.float32),
            pltpu.SemaphoreType.DMA,
        ],
    )
    def gather_k(table_hbm, idx_hbm, out_hbm, idx_v, rows_v, sem):
        wid = lax.axis_index("s") * info.num_cores + lax.axis_index("c")
        base = wid * b_per_w

        def step(i, _):
            off = base + i * chunk
            pltpu.sync_copy(idx_hbm.at[pl.ds(off, chunk)], idx_v)
            pltpu.async_copy(table_hbm.at[idx_v], rows_v, sem).wait()
            pltpu.sync_copy(rows_v, out_hbm.at[pl.ds(off, chunk)])
            return 0

        lax.fori_loop(0, n_chunks, step, 0)

    return gather_k(table, idx)


def kernel(z, W_proj, b_proj, embed_w):
    B, C, H, W = z.shape
    D = W_proj.shape[0]
    K = embed_w.shape[0]
    HW = H * W
    R = 256
    KB = 512
    HB = 4096

    zf = z.reshape(B, C, HW)
    bp = b_proj.reshape(D, 1)

    ze_t, ind3 = pl.pallas_call(
        functools.partial(_argmin_body, kb=KB, hb=HB),
        grid=(B, HW // R),
        in_specs=[
            pl.BlockSpec((1, C, R), lambda b, j: (b, 0, j)),
            pl.BlockSpec((D, C), lambda b, j: (0, 0)),
            pl.BlockSpec((D, 1), lambda b, j: (0, 0)),
            pl.BlockSpec((K, D), lambda b, j: (0, 0)),
        ],
        out_specs=[
            pl.BlockSpec((1, D, R), lambda b, j: (b, 0, j)),
            pl.BlockSpec((1, 1, R), lambda b, j: (b, 0, j)),
        ],
        out_shape=[
            jax.ShapeDtypeStruct((B, D, HW), jnp.float32),
            jax.ShapeDtypeStruct((B, 1, HW), jnp.int32),
        ],
        scratch_shapes=[
            pltpu.VMEM((K, D), jnp.bfloat16),
            pltpu.VMEM((K, 1), jnp.float32),
        ],
    )(zf, W_proj, bp, embed_w)

    ind_flat = ind3.reshape(B * HW)
    zq_rows = _sc_gather(embed_w, ind_flat)
    zq_t = zq_rows.reshape(B, HW, D).transpose(0, 2, 1)

    n_cells = B * (HW // R)
    zq_st, diffp = pl.pallas_call(
        functools.partial(_combine_body, n_total=float(B * HW * D),
                          n_cells=n_cells),
        grid=(B, HW // R),
        in_specs=[
            pl.BlockSpec((1, D, R), lambda b, j: (b, 0, j)),
            pl.BlockSpec((1, D, R), lambda b, j: (b, 0, j)),
        ],
        out_specs=[
            pl.BlockSpec((1, D, R), lambda b, j: (b, 0, j)),
            pl.BlockSpec((1, 1), lambda b, j: (0, 0)),
        ],
        out_shape=[
            jax.ShapeDtypeStruct((B, D, HW), jnp.float32),
            jax.ShapeDtypeStruct((1, 1), jnp.float32),
        ],
        scratch_shapes=[pltpu.SMEM((1,), jnp.float32)],
    )(ze_t, zq_t)

    z_q = zq_st.reshape(B, D, H, W)
    diff = diffp.reshape(())
    ind = ind3.reshape(B, H, W)
    return z_q, diff, ind


# speedup vs baseline: 1.2785x; 1.2039x over previous
import functools

import jax
import jax.numpy as jnp
from jax import lax
from jax.experimental import pallas as pl
from jax.experimental.pallas import tpu as pltpu
from jax.experimental.pallas import tpu_sc as plsc


def _argmin_body(z_ref, w_ref, b_ref, e_ref, ze_ref, ind_ref,
                 ebd_ref, e2_ref, *, kb, hb):
    first = (pl.program_id(0) == 0) & (pl.program_id(1) == 0)

    @pl.when(first)
    def _():
        e = e_ref[...]
        ebd_ref[...] = (2.0 * e).astype(jnp.bfloat16)
        e2_ref[...] = jnp.sum(e * e, axis=1, keepdims=True)

    zb = z_ref[0]
    w = w_ref[...]
    ze = lax.dot_general(w, zb, (((1,), (0,)), ((), ())),
                         preferred_element_type=jnp.float32) + b_ref[...]
    r = ze.shape[1]
    k_total = e_ref.shape[0]
    f2 = jnp.sum(ze * ze, axis=0, keepdims=True)
    ze_bf = ze.astype(jnp.bfloat16)
    iota = lax.broadcasted_iota(jnp.int32, (kb, r), 0)
    ns = hb // kb

    def inner(k, carry):
        lv, li = carry
        eb = ebd_ref[pl.ds(k * kb, kb), :]
        s2 = lax.dot_general(eb, ze_bf, (((1,), (0,)), ((), ())),
                             preferred_element_type=jnp.float32)
        e2 = e2_ref[pl.ds(k * kb, kb), :]
        dist = (f2 - s2) + e2
        m = jnp.min(dist, axis=0, keepdims=True)
        ci = jnp.min(jnp.where(dist == m, iota, jnp.int32(2**30)),
                     axis=0, keepdims=True) + k * kb
        upd = (m < lv) | ((m == lv) & (ci < li))
        return jnp.where(upd, m, lv), jnp.where(upd, ci, li)

    def outer(h, carry):
        bv, bi = carry
        lv0 = jnp.full((1, r), jnp.inf, jnp.float32)
        li0 = jnp.full((1, r), jnp.int32(2**30))
        lv, li = lax.fori_loop(h * ns, (h + 1) * ns, inner, (lv0, li0))
        take = lv < bv
        lvq = lv.astype(jnp.bfloat16).astype(jnp.float32)
        return jnp.where(take, lvq, bv), jnp.where(take, li, bi)

    bv0 = jnp.full((1, r), jnp.inf, jnp.float32)
    bi0 = jnp.zeros((1, r), jnp.int32)
    _, bi = lax.fori_loop(0, k_total // hb, outer, (bv0, bi0))
    ze_ref[0] = ze
    ind_ref[0] = bi


def _combine_body(ze_ref, zq_ref, out_ref, diff_ref, acc_ref, *, n_total, n_cells):
    ze = ze_ref[0]
    zq = zq_ref[0]
    delta = zq - ze
    out_ref[0] = ze + delta
    part = jnp.sum(delta * delta)
    cell = pl.program_id(0) * pl.num_programs(1) + pl.program_id(1)

    @pl.when(cell == 0)
    def _():
        acc_ref[0] = part

    @pl.when(cell != 0)
    def _():
        acc_ref[0] += part

    @pl.when(cell == n_cells - 1)
    def _():
        m = acc_ref[0] / n_total
        diff_ref[...] = jnp.full((1, 1), (0.25 * m + m) * 10.0, jnp.float32)


def _sc_gather(table, idx):
    n = idx.shape[0]
    d = table.shape[1]
    info = plsc.get_sparse_core_info()
    nw = info.num_cores * info.num_subcores
    b_per_w = n // nw
    chunk = 128
    n_chunks = b_per_w // chunk
    mesh = plsc.VectorSubcoreMesh(core_axis_name="c", subcore_axis_name="s")

    @functools.partial(
        pl.kernel, mesh=mesh,
        compiler_params=pltpu.CompilerParams(use_tc_tiling_on_sc=False),
        out_type=jax.ShapeDtypeStruct((n, d), jnp.float32),
        scratch_types=[
            pltpu.VMEM((chunk,), jnp.int32),
            pltpu.VMEM((chunk, d), jnp.float32),
            pltpu.SemaphoreType.DMA,
        ],
    )
    def gather_k(table_hbm, idx_hbm, out_hbm, idx_v, rows_v, sem):
        wid = lax.axis_index("s") * info.num_cores + lax.axis_index("c")
        base = wid * b_per_w

        def step(i, _):
            off = base + i * chunk
            pltpu.sync_copy(idx_hbm.at[pl.ds(off, chunk)], idx_v)
            pltpu.async_copy(table_hbm.at[idx_v], rows_v, sem).wait()
            pltpu.sync_copy(rows_v, out_hbm.at[pl.ds(off, chunk)])
            return 0

        lax.fori_loop(0, n_chunks, step, 0)

    return gather_k(table, idx)


def kernel(z, W_proj, b_proj, embed_w):
    B, C, H, W = z.shape
    D = W_proj.shape[0]
    K = embed_w.shape[0]
    HW = H * W
    R = 512
    KB = 512
    HB = 4096

    zf = z.reshape(B, C, HW)
    bp = b_proj.reshape(D, 1)

    ze_t, ind3 = pl.pallas_call(
        functools.partial(_argmin_body, kb=KB, hb=HB),
        grid=(B, HW // R),
        in_specs=[
            pl.BlockSpec((1, C, R), lambda b, j: (b, 0, j)),
            pl.BlockSpec((D, C), lambda b, j: (0, 0)),
            pl.BlockSpec((D, 1), lambda b, j: (0, 0)),
            pl.BlockSpec((K, D), lambda b, j: (0, 0)),
        ],
        out_specs=[
            pl.BlockSpec((1, D, R), lambda b, j: (b, 0, j)),
            pl.BlockSpec((1, 1, R), lambda b, j: (b, 0, j)),
        ],
        out_shape=[
            jax.ShapeDtypeStruct((B, D, HW), jnp.float32),
            jax.ShapeDtypeStruct((B, 1, HW), jnp.int32),
        ],
        scratch_shapes=[
            pltpu.VMEM((K, D), jnp.bfloat16),
            pltpu.VMEM((K, 1), jnp.float32),
        ],
    )(zf, W_proj, bp, embed_w)

    ind_flat = ind3.reshape(B * HW)
    zq_rows = _sc_gather(embed_w, ind_flat)
    zq_t = zq_rows.reshape(B, HW, D).transpose(0, 2, 1)

    n_cells = B * (HW // R)
    zq_st, diffp = pl.pallas_call(
        functools.partial(_combine_body, n_total=float(B * HW * D),
                          n_cells=n_cells),
        grid=(B, HW // R),
        in_specs=[
            pl.BlockSpec((1, D, R), lambda b, j: (b, 0, j)),
            pl.BlockSpec((1, D, R), lambda b, j: (b, 0, j)),
        ],
        out_specs=[
            pl.BlockSpec((1, D, R), lambda b, j: (b, 0, j)),
            pl.BlockSpec((1, 1), lambda b, j: (0, 0)),
        ],
        out_shape=[
            jax.ShapeDtypeStruct((B, D, HW), jnp.float32),
            jax.ShapeDtypeStruct((1, 1), jnp.float32),
        ],
        scratch_shapes=[pltpu.SMEM((1,), jnp.float32)],
    )(ze_t, zq_t)

    z_q = zq_st.reshape(B, D, H, W)
    diff = diffp.reshape(())
    ind = ind3.reshape(B, H, W)
    return z_q, diff, ind


# speedup vs baseline: 1.5036x; 1.1760x over previous
import functools

import jax
import jax.numpy as jnp
from jax import lax
from jax.experimental import pallas as pl
from jax.experimental.pallas import tpu as pltpu
from jax.experimental.pallas import tpu_sc as plsc


def _argmin_body(z_ref, w_ref, b_ref, e_ref, ze_ref, ind_ref,
                 ebd_ref, e2_ref, *, kb, hb):
    first = (pl.program_id(0) == 0) & (pl.program_id(1) == 0)

    @pl.when(first)
    def _():
        e = e_ref[...]
        ebd_ref[...] = (2.0 * e).astype(jnp.bfloat16)
        e2_ref[...] = jnp.sum(e * e, axis=1, keepdims=True)

    zb = z_ref[0]
    w = w_ref[...]
    ze = lax.dot_general(w, zb, (((1,), (0,)), ((), ())),
                         preferred_element_type=jnp.float32) + b_ref[...]
    r = ze.shape[1]
    k_total = e_ref.shape[0]
    f2 = jnp.sum(ze * ze, axis=0, keepdims=True)
    ze_bf = ze.astype(jnp.bfloat16)
    iota = lax.broadcasted_iota(jnp.int32, (kb, r), 0)
    ns = hb // kb

    def inner(k, carry):
        lv, li = carry
        eb = ebd_ref[pl.ds(k * kb, kb), :]
        s2 = lax.dot_general(eb, ze_bf, (((1,), (0,)), ((), ())),
                             preferred_element_type=jnp.float32)
        e2 = e2_ref[pl.ds(k * kb, kb), :]
        dist = (f2 - s2) + e2
        m = jnp.min(dist, axis=0, keepdims=True)
        ci = jnp.min(jnp.where(dist == m, iota, jnp.int32(2**30)),
                     axis=0, keepdims=True) + k * kb
        upd = (m < lv) | ((m == lv) & (ci < li))
        return jnp.where(upd, m, lv), jnp.where(upd, ci, li)

    def outer(h, carry):
        bv, bi = carry
        lv0 = jnp.full((1, r), jnp.inf, jnp.float32)
        li0 = jnp.full((1, r), jnp.int32(2**30))
        lv, li = lax.fori_loop(h * ns, (h + 1) * ns, inner, (lv0, li0))
        take = lv < bv
        lvq = lv.astype(jnp.bfloat16).astype(jnp.float32)
        return jnp.where(take, lvq, bv), jnp.where(take, li, bi)

    bv0 = jnp.full((1, r), jnp.inf, jnp.float32)
    bi0 = jnp.zeros((1, r), jnp.int32)
    _, bi = lax.fori_loop(0, k_total // hb, outer, (bv0, bi0))
    ze_ref[0] = ze
    ind_ref[0] = bi


def _combine_body(ze_ref, zq_ref, out_ref, diff_ref, acc_ref, *, n_total, n_cells):
    ze = ze_ref[0]
    zq = zq_ref[0]
    delta = zq - ze
    out_ref[0] = ze + delta
    part = jnp.sum(delta * delta)
    cell = pl.program_id(0) * pl.num_programs(1) + pl.program_id(1)

    @pl.when(cell == 0)
    def _():
        acc_ref[0] = part

    @pl.when(cell != 0)
    def _():
        acc_ref[0] += part

    @pl.when(cell == n_cells - 1)
    def _():
        m = acc_ref[0] / n_total
        diff_ref[...] = jnp.full((1, 1), (0.25 * m + m) * 10.0, jnp.float32)


def _sc_gather(table, idx):
    n = idx.shape[0]
    d = table.shape[1]
    info = plsc.get_sparse_core_info()
    nw = info.num_cores * info.num_subcores
    b_per_w = n // nw
    chunk = 128
    n_chunks = b_per_w // chunk
    mesh = plsc.VectorSubcoreMesh(core_axis_name="c", subcore_axis_name="s")

    @functools.partial(
        pl.kernel, mesh=mesh,
        compiler_params=pltpu.CompilerParams(use_tc_tiling_on_sc=False),
        out_type=jax.ShapeDtypeStruct((n, d), jnp.float32),
        scratch_types=[
            pltpu.VMEM((chunk,), jnp.int32),
            pltpu.VMEM((chunk, d), jnp.float32),
            pltpu.SemaphoreType.DMA,
        ],
    )
    def gather_k(table_hbm, idx_hbm, out_hbm, idx_v, rows_v, sem):
        wid = lax.axis_index("s") * info.num_cores + lax.axis_index("c")
        base = wid * b_per_w

        def step(i, _):
            off = base + i * chunk
            pltpu.sync_copy(idx_hbm.at[pl.ds(off, chunk)], idx_v)
            pltpu.async_copy(table_hbm.at[idx_v], rows_v, sem).wait()
            pltpu.sync_copy(rows_v, out_hbm.at[pl.ds(off, chunk)])
            return 0

        lax.fori_loop(0, n_chunks, step, 0)

    return gather_k(table, idx)


def kernel(z, W_proj, b_proj, embed_w):
    B, C, H, W = z.shape
    D = W_proj.shape[0]
    K = embed_w.shape[0]
    HW = H * W
    R = 512
    KB = 1024
    HB = 4096

    zf = z.reshape(B, C, HW)
    bp = b_proj.reshape(D, 1)

    ze_t, ind3 = pl.pallas_call(
        functools.partial(_argmin_body, kb=KB, hb=HB),
        grid=(B, HW // R),
        in_specs=[
            pl.BlockSpec((1, C, R), lambda b, j: (b, 0, j)),
            pl.BlockSpec((D, C), lambda b, j: (0, 0)),
            pl.BlockSpec((D, 1), lambda b, j: (0, 0)),
            pl.BlockSpec((K, D), lambda b, j: (0, 0)),
        ],
        out_specs=[
            pl.BlockSpec((1, D, R), lambda b, j: (b, 0, j)),
            pl.BlockSpec((1, 1, R), lambda b, j: (b, 0, j)),
        ],
        out_shape=[
            jax.ShapeDtypeStruct((B, D, HW), jnp.float32),
            jax.ShapeDtypeStruct((B, 1, HW), jnp.int32),
        ],
        scratch_shapes=[
            pltpu.VMEM((K, D), jnp.bfloat16),
            pltpu.VMEM((K, 1), jnp.float32),
        ],
    )(zf, W_proj, bp, embed_w)

    ind_flat = ind3.reshape(B * HW)
    zq_rows = _sc_gather(embed_w, ind_flat)
    zq_t = zq_rows.reshape(B, HW, D).transpose(0, 2, 1)

    n_cells = B * (HW // R)
    zq_st, diffp = pl.pallas_call(
        functools.partial(_combine_body, n_total=float(B * HW * D),
                          n_cells=n_cells),
        grid=(B, HW // R),
        in_specs=[
            pl.BlockSpec((1, D, R), lambda b, j: (b, 0, j)),
            pl.BlockSpec((1, D, R), lambda b, j: (b, 0, j)),
        ],
        out_specs=[
            pl.BlockSpec((1, D, R), lambda b, j: (b, 0, j)),
            pl.BlockSpec((1, 1), lambda b, j: (0, 0)),
        ],
        out_shape=[
            jax.ShapeDtypeStruct((B, D, HW), jnp.float32),
            jax.ShapeDtypeStruct((1, 1), jnp.float32),
        ],
        scratch_shapes=[pltpu.SMEM((1,), jnp.float32)],
    )(ze_t, zq_t)

    z_q = zq_st.reshape(B, D, H, W)
    diff = diffp.reshape(())
    ind = ind3.reshape(B, H, W)
    return z_q, diff, ind


# speedup vs baseline: 1.6028x; 1.0660x over previous
import functools

import jax
import jax.numpy as jnp
from jax import lax
from jax.experimental import pallas as pl
from jax.experimental.pallas import tpu as pltpu
from jax.experimental.pallas import tpu_sc as plsc


def _argmin_body(z_ref, w_ref, b_ref, e_ref, ze_ref, ind_ref,
                 ebd_ref, e2_ref, *, kb, hb):
    first = (pl.program_id(0) == 0) & (pl.program_id(1) == 0)

    @pl.when(first)
    def _():
        e = e_ref[...]
        ebd_ref[...] = (2.0 * e).astype(jnp.bfloat16)
        e2_ref[...] = jnp.sum(e * e, axis=1, keepdims=True)

    zb = z_ref[0]
    w = w_ref[...]
    ze = lax.dot_general(w, zb, (((1,), (0,)), ((), ())),
                         preferred_element_type=jnp.float32) + b_ref[...]
    r = ze.shape[1]
    k_total = e_ref.shape[0]
    f2 = jnp.sum(ze * ze, axis=0, keepdims=True)
    ze_bf = ze.astype(jnp.bfloat16)
    iota = lax.broadcasted_iota(jnp.int32, (kb, r), 0)
    ns = hb // kb

    def inner(k, carry):
        lv, li = carry
        eb = ebd_ref[pl.ds(k * kb, kb), :]
        s2 = lax.dot_general(eb, ze_bf, (((1,), (0,)), ((), ())),
                             preferred_element_type=jnp.float32)
        e2 = e2_ref[pl.ds(k * kb, kb), :]
        dist = (f2 - s2) + e2
        m = jnp.min(dist, axis=0, keepdims=True)
        ci = jnp.min(jnp.where(dist == m, iota, jnp.int32(2**30)),
                     axis=0, keepdims=True) + k * kb
        upd = (m < lv) | ((m == lv) & (ci < li))
        return jnp.where(upd, m, lv), jnp.where(upd, ci, li)

    def outer(h, carry):
        bv, bi = carry
        lv0 = jnp.full((1, r), jnp.inf, jnp.float32)
        li0 = jnp.full((1, r), jnp.int32(2**30))
        lv, li = lax.fori_loop(h * ns, (h + 1) * ns, inner, (lv0, li0))
        take = lv < bv
        lvq = lv.astype(jnp.bfloat16).astype(jnp.float32)
        return jnp.where(take, lvq, bv), jnp.where(take, li, bi)

    bv0 = jnp.full((1, r), jnp.inf, jnp.float32)
    bi0 = jnp.zeros((1, r), jnp.int32)
    _, bi = lax.fori_loop(0, k_total // hb, outer, (bv0, bi0))
    ze_ref[0] = ze
    ind_ref[0] = bi


def _combine_body(ze_ref, zq_ref, out_ref, diff_ref, acc_ref, *, n_total, n_cells):
    ze = ze_ref[0]
    zq = zq_ref[0]
    delta = zq - ze
    out_ref[0] = ze + delta
    part = jnp.sum(delta * delta)
    cell = pl.program_id(0) * pl.num_programs(1) + pl.program_id(1)

    @pl.when(cell == 0)
    def _():
        acc_ref[0] = part

    @pl.when(cell != 0)
    def _():
        acc_ref[0] += part

    @pl.when(cell == n_cells - 1)
    def _():
        m = acc_ref[0] / n_total
        diff_ref[...] = jnp.full((1, 1), (0.25 * m + m) * 10.0, jnp.float32)


def _sc_gather(table, idx):
    n = idx.shape[0]
    d = table.shape[1]
    info = plsc.get_sparse_core_info()
    nw = info.num_cores * info.num_subcores
    b_per_w = n // nw
    chunk = 128
    n_chunks = b_per_w // chunk
    mesh = plsc.VectorSubcoreMesh(core_axis_name="c", subcore_axis_name="s")

    @functools.partial(
        pl.kernel, mesh=mesh,
        compiler_params=pltpu.CompilerParams(use_tc_tiling_on_sc=False),
        out_type=jax.ShapeDtypeStruct((n, d), jnp.float32),
        scratch_types=[
            pltpu.VMEM((chunk,), jnp.int32),
            pltpu.VMEM((chunk, d), jnp.float32),
            pltpu.SemaphoreType.DMA,
        ],
    )
    def gather_k(table_hbm, idx_hbm, out_hbm, idx_v, rows_v, sem):
        wid = lax.axis_index("s") * info.num_cores + lax.axis_index("c")
        base = wid * b_per_w

        def step(i, _):
            off = base + i * chunk
            pltpu.sync_copy(idx_hbm.at[pl.ds(off, chunk)], idx_v)
            pltpu.async_copy(table_hbm.at[idx_v], rows_v, sem).wait()
            pltpu.sync_copy(rows_v, out_hbm.at[pl.ds(off, chunk)])
            return 0

        lax.fori_loop(0, n_chunks, step, 0)

    return gather_k(table, idx)


def kernel(z, W_proj, b_proj, embed_w):
    B, C, H, W = z.shape
    D = W_proj.shape[0]
    K = embed_w.shape[0]
    HW = H * W
    R = 512
    KB = 2048
    HB = 4096

    zf = z.reshape(B, C, HW)
    bp = b_proj.reshape(D, 1)

    ze_t, ind3 = pl.pallas_call(
        functools.partial(_argmin_body, kb=KB, hb=HB),
        grid=(B, HW // R),
        in_specs=[
            pl.BlockSpec((1, C, R), lambda b, j: (b, 0, j)),
            pl.BlockSpec((D, C), lambda b, j: (0, 0)),
            pl.BlockSpec((D, 1), lambda b, j: (0, 0)),
            pl.BlockSpec((K, D), lambda b, j: (0, 0)),
        ],
        out_specs=[
            pl.BlockSpec((1, D, R), lambda b, j: (b, 0, j)),
            pl.BlockSpec((1, 1, R), lambda b, j: (b, 0, j)),
        ],
        out_shape=[
            jax.ShapeDtypeStruct((B, D, HW), jnp.float32),
            jax.ShapeDtypeStruct((B, 1, HW), jnp.int32),
        ],
        scratch_shapes=[
            pltpu.VMEM((K, D), jnp.bfloat16),
            pltpu.VMEM((K, 1), jnp.float32),
        ],
    )(zf, W_proj, bp, embed_w)

    ind_flat = ind3.reshape(B * HW)
    zq_rows = _sc_gather(embed_w, ind_flat)
    zq_t = zq_rows.reshape(B, HW, D).transpose(0, 2, 1)

    n_cells = B * (HW // R)
    zq_st, diffp = pl.pallas_call(
        functools.partial(_combine_body, n_total=float(B * HW * D),
                          n_cells=n_cells),
        grid=(B, HW // R),
        in_specs=[
            pl.BlockSpec((1, D, R), lambda b, j: (b, 0, j)),
            pl.BlockSpec((1, D, R), lambda b, j: (b, 0, j)),
        ],
        out_specs=[
            pl.BlockSpec((1, D, R), lambda b, j: (b, 0, j)),
            pl.BlockSpec((1, 1), lambda b, j: (0, 0)),
        ],
        out_shape=[
            jax.ShapeDtypeStruct((B, D, HW), jnp.float32),
            jax.ShapeDtypeStruct((1, 1), jnp.float32),
        ],
        scratch_shapes=[pltpu.SMEM((1,), jnp.float32)],
    )(ze_t, zq_t)

    z_q = zq_st.reshape(B, D, H, W)
    diff = diffp.reshape(())
    ind = ind3.reshape(B, H, W)
    return z_q, diff, ind


# speedup vs baseline: 1.6862x; 1.0521x over previous
import functools

import jax
import jax.numpy as jnp
from jax import lax
from jax.experimental import pallas as pl
from jax.experimental.pallas import tpu as pltpu
from jax.experimental.pallas import tpu_sc as plsc


def _argmin_body(z_ref, w_ref, b_ref, e_ref, ze_ref, ind_ref,
                 ebd_ref, e2_ref, *, kb, hb):
    first = (pl.program_id(0) == 0) & (pl.program_id(1) == 0)

    @pl.when(first)
    def _():
        e = e_ref[...]
        ebd_ref[...] = (2.0 * e).astype(jnp.bfloat16)
        e2_ref[...] = jnp.sum(e * e, axis=1, keepdims=True)

    zb = z_ref[0]
    w = w_ref[...]
    ze = lax.dot_general(w, zb, (((1,), (0,)), ((), ())),
                         preferred_element_type=jnp.float32) + b_ref[...]
    r = ze.shape[1]
    k_total = e_ref.shape[0]
    f2 = jnp.sum(ze * ze, axis=0, keepdims=True)
    ze_bf = ze.astype(jnp.bfloat16)
    iota = lax.broadcasted_iota(jnp.int32, (kb, r), 0)
    ns = hb // kb

    def inner(k, carry):
        lv, li = carry
        eb = ebd_ref[pl.ds(k * kb, kb), :]
        s2 = lax.dot_general(eb, ze_bf, (((1,), (0,)), ((), ())),
                             preferred_element_type=jnp.float32)
        e2 = e2_ref[pl.ds(k * kb, kb), :]
        dist = (f2 - s2) + e2
        m = jnp.min(dist, axis=0, keepdims=True)
        ci = jnp.min(jnp.where(dist == m, iota, jnp.int32(2**30)),
                     axis=0, keepdims=True) + k * kb
        upd = (m < lv) | ((m == lv) & (ci < li))
        return jnp.where(upd, m, lv), jnp.where(upd, ci, li)

    def outer(h, carry):
        bv, bi = carry
        lv0 = jnp.full((1, r), jnp.inf, jnp.float32)
        li0 = jnp.full((1, r), jnp.int32(2**30))
        lv, li = lax.fori_loop(h * ns, (h + 1) * ns, inner, (lv0, li0))
        take = lv < bv
        lvq = lv.astype(jnp.bfloat16).astype(jnp.float32)
        return jnp.where(take, lvq, bv), jnp.where(take, li, bi)

    bv0 = jnp.full((1, r), jnp.inf, jnp.float32)
    bi0 = jnp.zeros((1, r), jnp.int32)
    _, bi = lax.fori_loop(0, k_total // hb, outer, (bv0, bi0))
    ze_ref[0] = ze
    ind_ref[0] = bi


def _combine_body(ze_ref, zq_ref, out_ref, diff_ref, acc_ref, *, n_total, n_cells):
    ze = ze_ref[0]
    zq = zq_ref[0]
    delta = zq - ze
    out_ref[0] = ze + delta
    part = jnp.sum(delta * delta)
    cell = pl.program_id(0) * pl.num_programs(1) + pl.program_id(1)

    @pl.when(cell == 0)
    def _():
        acc_ref[0] = part

    @pl.when(cell != 0)
    def _():
        acc_ref[0] += part

    @pl.when(cell == n_cells - 1)
    def _():
        m = acc_ref[0] / n_total
        diff_ref[...] = jnp.full((1, 1), (0.25 * m + m) * 10.0, jnp.float32)


def _sc_gather(table, idx):
    n = idx.shape[0]
    d = table.shape[1]
    info = plsc.get_sparse_core_info()
    nw = info.num_cores * info.num_subcores
    b_per_w = n // nw
    chunk = 128
    n_chunks = b_per_w // chunk
    mesh = plsc.VectorSubcoreMesh(core_axis_name="c", subcore_axis_name="s")

    @functools.partial(
        pl.kernel, mesh=mesh,
        compiler_params=pltpu.CompilerParams(use_tc_tiling_on_sc=False),
        out_type=jax.ShapeDtypeStruct((n, d), jnp.float32),
        scratch_types=[
            pltpu.VMEM((chunk,), jnp.int32),
            pltpu.VMEM((chunk, d), jnp.float32),
            pltpu.SemaphoreType.DMA,
        ],
    )
    def gather_k(table_hbm, idx_hbm, out_hbm, idx_v, rows_v, sem):
        wid = lax.axis_index("s") * info.num_cores + lax.axis_index("c")
        base = wid * b_per_w

        def step(i, _):
            off = base + i * chunk
            pltpu.sync_copy(idx_hbm.at[pl.ds(off, chunk)], idx_v)
            pltpu.async_copy(table_hbm.at[idx_v], rows_v, sem).wait()
            pltpu.sync_copy(rows_v, out_hbm.at[pl.ds(off, chunk)])
            return 0

        lax.fori_loop(0, n_chunks, step, 0)

    return gather_k(table, idx)


def kernel(z, W_proj, b_proj, embed_w):
    B, C, H, W = z.shape
    D = W_proj.shape[0]
    K = embed_w.shape[0]
    HW = H * W
    R = 512
    KB = 4096
    HB = 4096

    zf = z.reshape(B, C, HW)
    bp = b_proj.reshape(D, 1)

    ze_t, ind3 = pl.pallas_call(
        functools.partial(_argmin_body, kb=KB, hb=HB),
        grid=(B, HW // R),
        in_specs=[
            pl.BlockSpec((1, C, R), lambda b, j: (b, 0, j)),
            pl.BlockSpec((D, C), lambda b, j: (0, 0)),
            pl.BlockSpec((D, 1), lambda b, j: (0, 0)),
            pl.BlockSpec((K, D), lambda b, j: (0, 0)),
        ],
        out_specs=[
            pl.BlockSpec((1, D, R), lambda b, j: (b, 0, j)),
            pl.BlockSpec((1, 1, R), lambda b, j: (b, 0, j)),
        ],
        out_shape=[
            jax.ShapeDtypeStruct((B, D, HW), jnp.float32),
            jax.ShapeDtypeStruct((B, 1, HW), jnp.int32),
        ],
        scratch_shapes=[
            pltpu.VMEM((K, D), jnp.bfloat16),
            pltpu.VMEM((K, 1), jnp.float32),
        ],
    )(zf, W_proj, bp, embed_w)

    ind_flat = ind3.reshape(B * HW)
    zq_rows = _sc_gather(embed_w, ind_flat)
    zq_t = zq_rows.reshape(B, HW, D).transpose(0, 2, 1)

    n_cells = B * (HW // R)
    zq_st, diffp = pl.pallas_call(
        functools.partial(_combine_body, n_total=float(B * HW * D),
                          n_cells=n_cells),
        grid=(B, HW // R),
        in_specs=[
            pl.BlockSpec((1, D, R), lambda b, j: (b, 0, j)),
            pl.BlockSpec((1, D, R), lambda b, j: (b, 0, j)),
        ],
        out_specs=[
            pl.BlockSpec((1, D, R), lambda b, j: (b, 0, j)),
            pl.BlockSpec((1, 1), lambda b, j: (0, 0)),
        ],
        out_shape=[
            jax.ShapeDtypeStruct((B, D, HW), jnp.float32),
            jax.ShapeDtypeStruct((1, 1), jnp.float32),
        ],
        scratch_shapes=[pltpu.SMEM((1,), jnp.float32)],
    )(ze_t, zq_t)

    z_q = zq_st.reshape(B, D, H, W)
    diff = diffp.reshape(())
    ind = ind3.reshape(B, H, W)
    return z_q, diff, ind


# speedup vs baseline: 1.7900x; 1.0615x over previous
import functools

import jax
import jax.numpy as jnp
from jax import lax
from jax.experimental import pallas as pl
from jax.experimental.pallas import tpu as pltpu
from jax.experimental.pallas import tpu_sc as plsc


def _argmin_body(z_ref, w_ref, b_ref, e_ref, ze_ref, ind_ref,
                 ebd_ref, e2_ref, *, kb, hb):
    first = (pl.program_id(0) == 0) & (pl.program_id(1) == 0)

    @pl.when(first)
    def _():
        e = e_ref[...]
        ebd_ref[...] = (2.0 * e).astype(jnp.bfloat16)
        e2_ref[...] = jnp.sum(e * e, axis=1, keepdims=True)

    zb = z_ref[0]
    w = w_ref[...]
    ze = lax.dot_general(w, zb, (((1,), (0,)), ((), ())),
                         preferred_element_type=jnp.float32) + b_ref[...]
    r = ze.shape[1]
    k_total = e_ref.shape[0]
    f2 = jnp.sum(ze * ze, axis=0, keepdims=True)
    ze_bf = ze.astype(jnp.bfloat16)
    iota = lax.broadcasted_iota(jnp.int32, (kb, r), 0)
    ns = hb // kb

    def inner(k, carry):
        lv, li = carry
        eb = ebd_ref[pl.ds(k * kb, kb), :]
        s2 = lax.dot_general(eb, ze_bf, (((1,), (0,)), ((), ())),
                             preferred_element_type=jnp.float32)
        e2 = e2_ref[pl.ds(k * kb, kb), :]
        dist = (f2 - s2) + e2
        m = jnp.min(dist, axis=0, keepdims=True)
        ci = jnp.min(jnp.where(dist == m, iota, jnp.int32(2**30)),
                     axis=0, keepdims=True) + k * kb
        upd = (m < lv) | ((m == lv) & (ci < li))
        return jnp.where(upd, m, lv), jnp.where(upd, ci, li)

    def outer(h, carry):
        bv, bi = carry
        lv0 = jnp.full((1, r), jnp.inf, jnp.float32)
        li0 = jnp.full((1, r), jnp.int32(2**30))
        lv, li = lax.fori_loop(h * ns, (h + 1) * ns, inner, (lv0, li0))
        take = lv < bv
        lvq = lv.astype(jnp.bfloat16).astype(jnp.float32)
        return jnp.where(take, lvq, bv), jnp.where(take, li, bi)

    bv0 = jnp.full((1, r), jnp.inf, jnp.float32)
    bi0 = jnp.zeros((1, r), jnp.int32)
    _, bi = lax.fori_loop(0, k_total // hb, outer, (bv0, bi0))
    ze_ref[0] = ze
    ind_ref[0] = bi


def _combine_body(ze_ref, zq_ref, out_ref, diff_ref, acc_ref, *, n_total, n_cells):
    ze = ze_ref[0]
    zq = zq_ref[0]
    delta = zq - ze
    out_ref[0] = ze + delta
    part = jnp.sum(delta * delta)
    cell = pl.program_id(0) * pl.num_programs(1) + pl.program_id(1)

    @pl.when(cell == 0)
    def _():
        acc_ref[0] = part

    @pl.when(cell != 0)
    def _():
        acc_ref[0] += part

    @pl.when(cell == n_cells - 1)
    def _():
        m = acc_ref[0] / n_total
        diff_ref[...] = jnp.full((1, 1), (0.25 * m + m) * 10.0, jnp.float32)


def _sc_gather(table, idx):
    n = idx.shape[0]
    d = table.shape[1]
    info = plsc.get_sparse_core_info()
    nw = info.num_cores * info.num_subcores
    b_per_w = n // nw
    chunk = 128
    n_chunks = b_per_w // chunk
    mesh = plsc.VectorSubcoreMesh(core_axis_name="c", subcore_axis_name="s")

    @functools.partial(
        pl.kernel, mesh=mesh,
        compiler_params=pltpu.CompilerParams(use_tc_tiling_on_sc=False),
        out_type=jax.ShapeDtypeStruct((n, d), jnp.float32),
        scratch_types=[
            pltpu.VMEM((chunk,), jnp.int32),
            pltpu.VMEM((chunk, d), jnp.float32),
            pltpu.SemaphoreType.DMA,
        ],
    )
    def gather_k(table_hbm, idx_hbm, out_hbm, idx_v, rows_v, sem):
        wid = lax.axis_index("s") * info.num_cores + lax.axis_index("c")
        base = wid * b_per_w

        def step(i, _):
            off = base + i * chunk
            pltpu.sync_copy(idx_hbm.at[pl.ds(off, chunk)], idx_v)
            pltpu.async_copy(table_hbm.at[idx_v], rows_v, sem).wait()
            pltpu.sync_copy(rows_v, out_hbm.at[pl.ds(off, chunk)])
            return 0

        lax.fori_loop(0, n_chunks, step, 0)

    return gather_k(table, idx)


def kernel(z, W_proj, b_proj, embed_w):
    B, C, H, W = z.shape
    D = W_proj.shape[0]
    K = embed_w.shape[0]
    HW = H * W
    R = 1024
    KB = 2048
    HB = 4096

    zf = z.reshape(B, C, HW)
    bp = b_proj.reshape(D, 1)

    ze_t, ind3 = pl.pallas_call(
        functools.partial(_argmin_body, kb=KB, hb=HB),
        grid=(B, HW // R),
        in_specs=[
            pl.BlockSpec((1, C, R), lambda b, j: (b, 0, j)),
            pl.BlockSpec((D, C), lambda b, j: (0, 0)),
            pl.BlockSpec((D, 1), lambda b, j: (0, 0)),
            pl.BlockSpec((K, D), lambda b, j: (0, 0)),
        ],
        out_specs=[
            pl.BlockSpec((1, D, R), lambda b, j: (b, 0, j)),
            pl.BlockSpec((1, 1, R), lambda b, j: (b, 0, j)),
        ],
        out_shape=[
            jax.ShapeDtypeStruct((B, D, HW), jnp.float32),
            jax.ShapeDtypeStruct((B, 1, HW), jnp.int32),
        ],
        scratch_shapes=[
            pltpu.VMEM((K, D), jnp.bfloat16),
            pltpu.VMEM((K, 1), jnp.float32),
        ],
    )(zf, W_proj, bp, embed_w)

    ind_flat = ind3.reshape(B * HW)
    zq_rows = _sc_gather(embed_w, ind_flat)
    zq_t = zq_rows.reshape(B, HW, D).transpose(0, 2, 1)

    n_cells = B * (HW // R)
    zq_st, diffp = pl.pallas_call(
        functools.partial(_combine_body, n_total=float(B * HW * D),
                          n_cells=n_cells),
        grid=(B, HW // R),
        in_specs=[
            pl.BlockSpec((1, D, R), lambda b, j: (b, 0, j)),
            pl.BlockSpec((1, D, R), lambda b, j: (b, 0, j)),
        ],
        out_specs=[
            pl.BlockSpec((1, D, R), lambda b, j: (b, 0, j)),
            pl.BlockSpec((1, 1), lambda b, j: (0, 0)),
        ],
        out_shape=[
            jax.ShapeDtypeStruct((B, D, HW), jnp.float32),
            jax.ShapeDtypeStruct((1, 1), jnp.float32),
        ],
        scratch_shapes=[pltpu.SMEM((1,), jnp.float32)],
    )(ze_t, zq_t)

    z_q = zq_st.reshape(B, D, H, W)
    diff = diffp.reshape(())
    ind = ind3.reshape(B, H, W)
    return z_q, diff, ind


# speedup vs baseline: 1.8237x; 1.0188x over previous
import functools

import jax
import jax.numpy as jnp
from jax import lax
from jax.experimental import pallas as pl
from jax.experimental.pallas import tpu as pltpu
from jax.experimental.pallas import tpu_sc as plsc


def _argmin_body(z_ref, w_ref, b_ref, e_ref, ze_ref, ind_ref,
                 ebd_ref, e2_ref, *, kb, hb):
    first = (pl.program_id(0) == 0) & (pl.program_id(1) == 0)

    @pl.when(first)
    def _():
        e = e_ref[...]
        ebd_ref[...] = (2.0 * e).astype(jnp.bfloat16)
        e2_ref[...] = jnp.sum(e * e, axis=1, keepdims=True)

    zb = z_ref[0]
    w = w_ref[...]
    ze = lax.dot_general(w, zb, (((1,), (0,)), ((), ())),
                         preferred_element_type=jnp.float32) + b_ref[...]
    r = ze.shape[1]
    k_total = e_ref.shape[0]
    f2 = jnp.sum(ze * ze, axis=0, keepdims=True)
    ze_bf = ze.astype(jnp.bfloat16)
    iota = lax.broadcasted_iota(jnp.int32, (kb, r), 0)
    ns = hb // kb

    def inner(k, carry):
        lv, li = carry
        eb = ebd_ref[pl.ds(k * kb, kb), :]
        s2 = lax.dot_general(eb, ze_bf, (((1,), (0,)), ((), ())),
                             preferred_element_type=jnp.float32)
        e2 = e2_ref[pl.ds(k * kb, kb), :]
        dist = (f2 - s2) + e2
        m = jnp.min(dist, axis=0, keepdims=True)
        ci = jnp.min(jnp.where(dist == m, iota, jnp.int32(2**30)),
                     axis=0, keepdims=True) + k * kb
        upd = (m < lv) | ((m == lv) & (ci < li))
        return jnp.where(upd, m, lv), jnp.where(upd, ci, li)

    def outer(h, carry):
        bv, bi = carry
        lv0 = jnp.full((1, r), jnp.inf, jnp.float32)
        li0 = jnp.full((1, r), jnp.int32(2**30))
        lv, li = lax.fori_loop(h * ns, (h + 1) * ns, inner, (lv0, li0))
        take = lv < bv
        lvq = lv.astype(jnp.bfloat16).astype(jnp.float32)
        return jnp.where(take, lvq, bv), jnp.where(take, li, bi)

    bv0 = jnp.full((1, r), jnp.inf, jnp.float32)
    bi0 = jnp.zeros((1, r), jnp.int32)
    _, bi = lax.fori_loop(0, k_total // hb, outer, (bv0, bi0))
    ze_ref[0] = ze
    ind_ref[0] = bi


def _combine_body(ze_ref, zq_ref, out_ref, diff_ref, acc_ref, *, n_total, n_cells):
    ze = ze_ref[0]
    zq = zq_ref[0]
    delta = zq - ze
    out_ref[0] = ze + delta
    part = jnp.sum(delta * delta)
    cell = pl.program_id(0) * pl.num_programs(1) + pl.program_id(1)

    @pl.when(cell == 0)
    def _():
        acc_ref[0] = part

    @pl.when(cell != 0)
    def _():
        acc_ref[0] += part

    @pl.when(cell == n_cells - 1)
    def _():
        m = acc_ref[0] / n_total
        diff_ref[...] = jnp.full((1, 1), (0.25 * m + m) * 10.0, jnp.float32)


def _sc_gather(table, idx):
    n = idx.shape[0]
    d = table.shape[1]
    info = plsc.get_sparse_core_info()
    nw = info.num_cores * info.num_subcores
    b_per_w = n // nw
    chunk = 128
    n_chunks = b_per_w // chunk
    mesh = plsc.VectorSubcoreMesh(core_axis_name="c", subcore_axis_name="s")

    @functools.partial(
        pl.kernel, mesh=mesh,
        compiler_params=pltpu.CompilerParams(use_tc_tiling_on_sc=False),
        out_type=jax.ShapeDtypeStruct((n, d), jnp.float32),
        scratch_types=[
            pltpu.VMEM((chunk,), jnp.int32),
            pltpu.VMEM((chunk, d), jnp.float32),
            pltpu.SemaphoreType.DMA,
        ],
    )
    def gather_k(table_hbm, idx_hbm, out_hbm, idx_v, rows_v, sem):
        wid = lax.axis_index("s") * info.num_cores + lax.axis_index("c")
        base = wid * b_per_w

        def step(i, _):
            off = base + i * chunk
            pltpu.sync_copy(idx_hbm.at[pl.ds(off, chunk)], idx_v)
            pltpu.async_copy(table_hbm.at[idx_v], rows_v, sem).wait()
            pltpu.sync_copy(rows_v, out_hbm.at[pl.ds(off, chunk)])
            return 0

        lax.fori_loop(0, n_chunks, step, 0)

    return gather_k(table, idx)


def kernel(z, W_proj, b_proj, embed_w):
    B, C, H, W = z.shape
    D = W_proj.shape[0]
    K = embed_w.shape[0]
    HW = H * W
    R = 1024
    KB = 4096
    HB = 4096

    zf = z.reshape(B, C, HW)
    bp = b_proj.reshape(D, 1)

    ze_t, ind3 = pl.pallas_call(
        functools.partial(_argmin_body, kb=KB, hb=HB),
        grid=(B, HW // R),
        in_specs=[
            pl.BlockSpec((1, C, R), lambda b, j: (b, 0, j)),
            pl.BlockSpec((D, C), lambda b, j: (0, 0)),
            pl.BlockSpec((D, 1), lambda b, j: (0, 0)),
            pl.BlockSpec((K, D), lambda b, j: (0, 0)),
        ],
        out_specs=[
            pl.BlockSpec((1, D, R), lambda b, j: (b, 0, j)),
            pl.BlockSpec((1, 1, R), lambda b, j: (b, 0, j)),
        ],
        out_shape=[
            jax.ShapeDtypeStruct((B, D, HW), jnp.float32),
            jax.ShapeDtypeStruct((B, 1, HW), jnp.int32),
        ],
        scratch_shapes=[
            pltpu.VMEM((K, D), jnp.bfloat16),
            pltpu.VMEM((K, 1), jnp.float32),
        ],
    )(zf, W_proj, bp, embed_w)

    ind_flat = ind3.reshape(B * HW)
    zq_rows = _sc_gather(embed_w, ind_flat)
    zq_t = zq_rows.reshape(B, HW, D).transpose(0, 2, 1)

    n_cells = B * (HW // R)
    zq_st, diffp = pl.pallas_call(
        functools.partial(_combine_body, n_total=float(B * HW * D),
                          n_cells=n_cells),
        grid=(B, HW // R),
        in_specs=[
            pl.BlockSpec((1, D, R), lambda b, j: (b, 0, j)),
            pl.BlockSpec((1, D, R), lambda b, j: (b, 0, j)),
        ],
        out_specs=[
            pl.BlockSpec((1, D, R), lambda b, j: (b, 0, j)),
            pl.BlockSpec((1, 1), lambda b, j: (0, 0)),
        ],
        out_shape=[
            jax.ShapeDtypeStruct((B, D, HW), jnp.float32),
            jax.ShapeDtypeStruct((1, 1), jnp.float32),
        ],
        scratch_shapes=[pltpu.SMEM((1,), jnp.float32)],
    )(ze_t, zq_t)

    z_q = zq_st.reshape(B, D, H, W)
    diff = diffp.reshape(())
    ind = ind3.reshape(B, H, W)
    return z_q, diff, ind
